# baseline (speedup 1.0000x reference)
import functools

import jax
import jax.numpy as jnp
import numpy as np
from jax import lax
from jax.experimental import pallas as pl
from jax.experimental.pallas import tpu as pltpu
from jax.experimental.pallas import tpu_sc as plsc

_L = 16
_OUT_CHUNKS = 4


def _make_scatter(BC, HW, P, NW):
  nplanes = BC // NW
  mesh = plsc.VectorSubcoreMesh(core_axis_name="c", subcore_axis_name="s")
  NC = mesh.num_cores
  CH = P // _OUT_CHUNKS

  @functools.partial(
      pl.kernel,
      out_type=jax.ShapeDtypeStruct((BC, P), jnp.float32),
      mesh=mesh,
      compiler_params=pltpu.CompilerParams(needs_layout_passes=False),
      scratch_types=[
          pltpu.VMEM((P,), jnp.float32),
          pltpu.VMEM((2, HW), jnp.int32),
          pltpu.VMEM((HW,), jnp.float32),
          pltpu.SemaphoreType.DMA,
          pltpu.SemaphoreType.DMA,
          pltpu.SemaphoreType.DMA,
          pltpu.SemaphoreType.DMA,
          pltpu.SemaphoreType.DMA,
          pltpu.SemaphoreType.DMA,
          pltpu.SemaphoreType.DMA,
      ],
  )
  def scatter_planes(vals_hbm, am_hbm, out_hbm, acc_v, am2_v, vals_v,
                     am_s0, am_s1, vals_s, o_s0, o_s1, o_s2, o_s3):
    wid = lax.axis_index("s") * NC + lax.axis_index("c")
    am_sems = (am_s0, am_s1)
    out_sems = (o_s0, o_s1, o_s2, o_s3)

    third = jnp.float32(1.0 / 3.0)
    zeros = jnp.zeros((_L,), jnp.float32)

    def zero_range(base, nvec):
      def zb(i, c):
        acc_v[pl.ds(base + i * _L, _L)] = zeros
        return c

      lax.fori_loop(0, nvec, zb, 0, unroll=8)

    pend_am = {0: pltpu.async_copy(am_hbm.at[wid], am2_v.at[0], am_s0)}
    pend_vals = pltpu.async_copy(vals_hbm.at[wid], vals_v, vals_s)
    zero_range(0, P // _L)

    for j in range(nplanes):
      buf = j % 2
      plane = j * NW + wid
      pend_am[buf].wait()
      if j + 1 < nplanes:
        nbuf = 1 - buf
        pend_am[nbuf] = pltpu.async_copy(
            am_hbm.at[(j + 1) * NW + wid], am2_v.at[nbuf], am_sems[nbuf])
      pend_vals.wait()

      @plsc.parallel_loop(0, HW, _L, unroll=8)
      def scat(i, _buf=buf):
        am = am2_v[_buf, pl.ds(i, _L)]
        v = vals_v[pl.ds(i, _L)]
        q = (jnp.right_shift(am, 5).astype(jnp.float32) * third).astype(
            jnp.int32)
        p = jnp.bitwise_and(q, P - 1)
        plsc.addupdate_scatter(acc_v, [p], v)

      if j + 1 < nplanes:
        pend_vals = pltpu.async_copy(
            vals_hbm.at[(j + 1) * NW + wid], vals_v, vals_s)

      out_d = [
          pltpu.async_copy(acc_v.at[pl.ds(k * CH, CH)],
                           out_hbm.at[plane, pl.ds(k * CH, CH)], out_sems[k])
          for k in range(2)
      ]
      for k in range(_OUT_CHUNKS):
        if k + 2 < _OUT_CHUNKS:
          out_d.append(
              pltpu.async_copy(acc_v.at[pl.ds((k + 2) * CH, CH)],
                               out_hbm.at[plane, pl.ds((k + 2) * CH, CH)],
                               out_sems[k + 2]))
        out_d[k].wait()
        if j + 1 < nplanes:
          zero_range(k * CH, CH // _L)

  return scatter_planes


def kernel(updates, argmax):
  B, H, W, C = updates.shape
  OH, OW = 2 * H, 2 * W
  HW = H * W
  P = OH * OW

  info = plsc.get_sparse_core_info()
  NW = info.num_cores * info.num_subcores

  scatter = _make_scatter(C, HW, P, NW)
  outs = []
  for b in range(B):
    vt = jnp.transpose(updates[b].reshape(HW, C))
    at = jnp.transpose(argmax[b].reshape(HW, C))
    ot = scatter(vt, at)
    outs.append(jnp.transpose(ot).reshape(OH, OW, C))
  return jnp.stack(outs)


# speedup vs baseline: 1.3116x; 1.3116x over previous
import functools

import jax
import jax.numpy as jnp
import numpy as np
from jax import lax
from jax.experimental import pallas as pl
from jax.experimental.pallas import tpu as pltpu
from jax.experimental.pallas import tpu_sc as plsc

_L = 16
_OUT_CHUNKS = 4


def _make_scatter(BC, HW, P, NW):
  nplanes = BC // NW
  mesh = plsc.VectorSubcoreMesh(core_axis_name="c", subcore_axis_name="s")
  NC = mesh.num_cores
  CH = P // _OUT_CHUNKS

  @functools.partial(
      pl.kernel,
      out_type=jax.ShapeDtypeStruct((BC, P), jnp.float32),
      mesh=mesh,
      compiler_params=pltpu.CompilerParams(needs_layout_passes=False),
      scratch_types=[
          pltpu.VMEM((P,), jnp.float32),
          pltpu.VMEM((2, HW), jnp.int32),
          pltpu.VMEM((HW,), jnp.float32),
          pltpu.SemaphoreType.DMA,
          pltpu.SemaphoreType.DMA,
          pltpu.SemaphoreType.DMA,
          pltpu.SemaphoreType.DMA,
          pltpu.SemaphoreType.DMA,
          pltpu.SemaphoreType.DMA,
          pltpu.SemaphoreType.DMA,
      ],
  )
  def scatter_planes(vals_hbm, am_hbm, out_hbm, acc_v, am2_v, vals_v,
                     am_s0, am_s1, vals_s, o_s0, o_s1, o_s2, o_s3):
    wid = lax.axis_index("s") * NC + lax.axis_index("c")
    am_sems = (am_s0, am_s1)
    out_sems = (o_s0, o_s1, o_s2, o_s3)

    third = jnp.float32(1.0 / 3.0)
    zeros = jnp.zeros((_L,), jnp.float32)

    def zero_range(base, nvec):
      def zb(i, c):
        acc_v[pl.ds(base + i * _L, _L)] = zeros
        return c

      lax.fori_loop(0, nvec, zb, 0, unroll=8)

    pend_am = {0: pltpu.async_copy(am_hbm.at[wid], am2_v.at[0], am_s0)}
    pend_vals = pltpu.async_copy(vals_hbm.at[wid], vals_v, vals_s)
    zero_range(0, P // _L)

    for j in range(nplanes):
      buf = j % 2
      plane = j * NW + wid
      pend_am[buf].wait()
      if j + 1 < nplanes:
        nbuf = 1 - buf
        pend_am[nbuf] = pltpu.async_copy(
            am_hbm.at[(j + 1) * NW + wid], am2_v.at[nbuf], am_sems[nbuf])
      pend_vals.wait()

      @plsc.parallel_loop(0, HW, _L, unroll=8)
      def scat(i, _buf=buf):
        am = am2_v[_buf, pl.ds(i, _L)]
        v = vals_v[pl.ds(i, _L)]
        q = (jnp.right_shift(am, 5).astype(jnp.float32) * third).astype(
            jnp.int32)
        p = jnp.bitwise_and(q, P - 1)
        plsc.addupdate_scatter(acc_v, [p], v)

      if j + 1 < nplanes:
        pend_vals = pltpu.async_copy(
            vals_hbm.at[(j + 1) * NW + wid], vals_v, vals_s)

      out_d = [
          pltpu.async_copy(acc_v.at[pl.ds(k * CH, CH)],
                           out_hbm.at[plane, pl.ds(k * CH, CH)], out_sems[k])
          for k in range(2)
      ]
      for k in range(_OUT_CHUNKS):
        if k + 2 < _OUT_CHUNKS:
          out_d.append(
              pltpu.async_copy(acc_v.at[pl.ds((k + 2) * CH, CH)],
                               out_hbm.at[plane, pl.ds((k + 2) * CH, CH)],
                               out_sems[k + 2]))
        out_d[k].wait()
        if j + 1 < nplanes:
          zero_range(k * CH, CH // _L)

  return scatter_planes


_BK = 2048
_PK = 2048


def _in_body(u_ref, a_ref, vt_ref, at_ref):
  vt_ref[...] = u_ref[0].T
  at_ref[...] = a_ref[0].T


def _make_in_t(B, HW, C, b):
  return pl.pallas_call(
      _in_body,
      grid=(HW // _BK,),
      in_specs=[
          pl.BlockSpec((1, _BK, C), lambda g, _b=b: (_b, g, 0)),
          pl.BlockSpec((1, _BK, C), lambda g, _b=b: (_b, g, 0)),
      ],
      out_specs=[
          pl.BlockSpec((C, _BK), lambda g: (0, g)),
          pl.BlockSpec((C, _BK), lambda g: (0, g)),
      ],
      out_shape=[
          jax.ShapeDtypeStruct((C, HW), jnp.float32),
          jax.ShapeDtypeStruct((C, HW), jnp.int32),
      ],
  )


def _out_body_first(s_ref, o_ref):
  o_ref[...] = s_ref[...].T[None]


def _out_body(big_ref, s_ref, o_ref):
  del big_ref
  o_ref[...] = s_ref[...].T[None]


def _make_out_t(B, P, C, b):
  out_spec = pl.BlockSpec((1, _PK, C), lambda g, _b=b: (_b, g, 0))
  src_spec = pl.BlockSpec((C, _PK), lambda g: (0, g))
  out_shape = jax.ShapeDtypeStruct((B, P, C), jnp.float32)
  if b == 0:
    return pl.pallas_call(
        _out_body_first,
        grid=(P // _PK,),
        in_specs=[src_spec],
        out_specs=out_spec,
        out_shape=out_shape,
    )
  return pl.pallas_call(
      _out_body,
      grid=(P // _PK,),
      in_specs=[pl.BlockSpec(memory_space=pl.ANY), src_spec],
      out_specs=out_spec,
      out_shape=out_shape,
      input_output_aliases={0: 0},
  )


def kernel(updates, argmax):
  B, H, W, C = updates.shape
  OH, OW = 2 * H, 2 * W
  HW = H * W
  P = OH * OW

  info = plsc.get_sparse_core_info()
  NW = info.num_cores * info.num_subcores

  u3 = updates.reshape(B, HW, C)
  a3 = argmax.reshape(B, HW, C)

  scatter = _make_scatter(C, HW, P, NW)
  big = None
  for b in range(B):
    vt, at = _make_in_t(B, HW, C, b)(u3, a3)
    ot = scatter(vt, at)
    if b == 0:
      big = _make_out_t(B, P, C, b)(ot)
    else:
      big = _make_out_t(B, P, C, b)(big, ot)
  return big.reshape(B, OH, OW, C)


# speedup vs baseline: 1.7807x; 1.3576x over previous
import functools

import jax
import jax.numpy as jnp
import numpy as np
from jax import lax
from jax.experimental import pallas as pl
from jax.experimental.pallas import tpu as pltpu
from jax.experimental.pallas import tpu_sc as plsc

_L = 16
_OUT_CHUNKS = 4


def _make_scatter(BC, HW, P, NW):
  nplanes = BC // NW
  mesh = plsc.VectorSubcoreMesh(core_axis_name="c", subcore_axis_name="s")
  NC = mesh.num_cores
  CH = P // _OUT_CHUNKS

  @functools.partial(
      pl.kernel,
      out_type=jax.ShapeDtypeStruct((BC, P), jnp.float32),
      mesh=mesh,
      compiler_params=pltpu.CompilerParams(needs_layout_passes=False),
      scratch_types=[
          pltpu.VMEM((P,), jnp.float32),
          pltpu.VMEM((2, HW), jnp.int32),
          pltpu.VMEM((HW,), jnp.float32),
          pltpu.SemaphoreType.DMA,
          pltpu.SemaphoreType.DMA,
          pltpu.SemaphoreType.DMA,
          pltpu.SemaphoreType.DMA,
          pltpu.SemaphoreType.DMA,
          pltpu.SemaphoreType.DMA,
          pltpu.SemaphoreType.DMA,
      ],
  )
  def scatter_planes(vals_hbm, am_hbm, out_hbm, acc_v, am2_v, vals_v,
                     am_s0, am_s1, vals_s, o_s0, o_s1, o_s2, o_s3):
    wid = lax.axis_index("s") * NC + lax.axis_index("c")
    am_sems = (am_s0, am_s1)
    out_sems = (o_s0, o_s1, o_s2, o_s3)

    third = jnp.float32(1.0 / 3.0)
    zeros = jnp.zeros((_L,), jnp.float32)

    def zero_range(base, nvec):
      def zb(i, c):
        acc_v[pl.ds(base + i * _L, _L)] = zeros
        return c

      lax.fori_loop(0, nvec, zb, 0, unroll=8)

    pend_am = {0: pltpu.async_copy(am_hbm.at[wid], am2_v.at[0], am_s0)}
    pend_vals = pltpu.async_copy(vals_hbm.at[wid], vals_v, vals_s)
    zero_range(0, P // _L)

    for j in range(nplanes):
      buf = j % 2
      plane = j * NW + wid
      pend_am[buf].wait()
      if j + 1 < nplanes:
        nbuf = 1 - buf
        pend_am[nbuf] = pltpu.async_copy(
            am_hbm.at[(j + 1) * NW + wid], am2_v.at[nbuf], am_sems[nbuf])
      pend_vals.wait()

      @plsc.parallel_loop(0, HW, _L, unroll=8)
      def scat(i, _buf=buf):
        am = am2_v[_buf, pl.ds(i, _L)]
        v = vals_v[pl.ds(i, _L)]
        q = (jnp.right_shift(am, 5).astype(jnp.float32) * third).astype(
            jnp.int32)
        p = jnp.bitwise_and(q, P - 1)
        plsc.addupdate_scatter(acc_v, [p], v)

      if j + 1 < nplanes:
        pend_vals = pltpu.async_copy(
            vals_hbm.at[(j + 1) * NW + wid], vals_v, vals_s)

      out_d = [
          pltpu.async_copy(acc_v.at[pl.ds(k * CH, CH)],
                           out_hbm.at[plane, pl.ds(k * CH, CH)], out_sems[k])
          for k in range(2)
      ]
      for k in range(_OUT_CHUNKS):
        if k + 2 < _OUT_CHUNKS:
          out_d.append(
              pltpu.async_copy(acc_v.at[pl.ds((k + 2) * CH, CH)],
                               out_hbm.at[plane, pl.ds((k + 2) * CH, CH)],
                               out_sems[k + 2]))
        out_d[k].wait()
        if j + 1 < nplanes:
          zero_range(k * CH, CH // _L)

  return scatter_planes


_BK = 2048
_PK = 2048


def _in_body(u_ref, a_ref, vt_ref, at_ref):
  vt_ref[...] = u_ref[0].T
  at_ref[...] = a_ref[0].T


def _make_in_t(B, HW, C, b):
  return pl.pallas_call(
      _in_body,
      grid=(HW // _BK,),
      in_specs=[
          pl.BlockSpec((1, _BK, C), lambda g, _b=b: (_b, g, 0)),
          pl.BlockSpec((1, _BK, C), lambda g, _b=b: (_b, g, 0)),
      ],
      out_specs=[
          pl.BlockSpec((C, _BK), lambda g: (0, g)),
          pl.BlockSpec((C, _BK), lambda g: (0, g)),
      ],
      out_shape=[
          jax.ShapeDtypeStruct((C, HW), jnp.float32),
          jax.ShapeDtypeStruct((C, HW), jnp.int32),
      ],
  )


def _out_body_first(s_ref, o_ref):
  o_ref[...] = s_ref[...].T[None]


def _out_body(big_ref, s_ref, o_ref):
  del big_ref
  o_ref[...] = s_ref[...].T[None]


def _make_out_t(B, P, C, b):
  out_spec = pl.BlockSpec((1, _PK, C), lambda g, _b=b: (_b, g, 0))
  src_spec = pl.BlockSpec((C, _PK), lambda g: (0, g))
  out_shape = jax.ShapeDtypeStruct((B, P, C), jnp.float32)
  if b == 0:
    return pl.pallas_call(
        _out_body_first,
        grid=(P // _PK,),
        in_specs=[src_spec],
        out_specs=out_spec,
        out_shape=out_shape,
    )
  return pl.pallas_call(
      _out_body,
      grid=(P // _PK,),
      in_specs=[pl.BlockSpec(memory_space=pl.ANY), src_spec],
      out_specs=out_spec,
      out_shape=out_shape,
      input_output_aliases={0: 0},
  )


def kernel(updates, argmax):
  B, H, W, C = updates.shape
  OH, OW = 2 * H, 2 * W
  HW = H * W
  P = OH * OW

  info = plsc.get_sparse_core_info()
  NW = info.num_cores * info.num_subcores

  vals_t = jnp.transpose(updates.reshape(B, HW, C), (0, 2, 1)).reshape(B * C, HW)
  am_t = jnp.transpose(argmax.reshape(B, HW, C), (0, 2, 1)).reshape(B * C, HW)

  out_t = _make_scatter(B * C, HW, P, NW)(vals_t, am_t)

  one = vals_t[0, 0] * jnp.float32(0.0) + jnp.float32(1.0)
  out = jnp.transpose(out_t.reshape(B, C, P), (0, 2, 1)) * one
  return out.reshape(B, OH, OW, C)


# speedup vs baseline: 2.2815x; 1.2813x over previous
import functools

import jax
import jax.numpy as jnp
import numpy as np
from jax import lax
from jax.experimental import pallas as pl
from jax.experimental.pallas import tpu as pltpu
from jax.experimental.pallas import tpu_sc as plsc

_L = 16
_OUT_CHUNKS = 4


def _make_scatter(BC, HW, P, NW):
  nplanes = BC // NW
  mesh = plsc.VectorSubcoreMesh(core_axis_name="c", subcore_axis_name="s")
  NC = mesh.num_cores
  CH = P // _OUT_CHUNKS

  @functools.partial(
      pl.kernel,
      out_type=jax.ShapeDtypeStruct((BC, P), jnp.float32),
      mesh=mesh,
      compiler_params=pltpu.CompilerParams(
          needs_layout_passes=False, use_tc_tiling_on_sc=True),
      scratch_types=[
          pltpu.VMEM((P,), jnp.float32),
          pltpu.VMEM((2, HW), jnp.int32),
          pltpu.VMEM((HW,), jnp.float32),
          pltpu.SemaphoreType.DMA,
          pltpu.SemaphoreType.DMA,
          pltpu.SemaphoreType.DMA,
          pltpu.SemaphoreType.DMA,
          pltpu.SemaphoreType.DMA,
          pltpu.SemaphoreType.DMA,
          pltpu.SemaphoreType.DMA,
      ],
  )
  def scatter_planes(vals_hbm, am_hbm, out_hbm, acc_v, am2_v, vals_v,
                     am_s0, am_s1, vals_s, o_s0, o_s1, o_s2, o_s3):
    wid = lax.axis_index("s") * NC + lax.axis_index("c")
    am_sems = (am_s0, am_s1)
    out_sems = (o_s0, o_s1, o_s2, o_s3)

    third = jnp.float32(1.0 / 3.0)
    zeros = jnp.zeros((_L,), jnp.float32)

    def zero_range(base, nvec):
      def zb(i, c):
        acc_v[pl.ds(base + i * _L, _L)] = zeros
        return c

      lax.fori_loop(0, nvec, zb, 0, unroll=8)

    pend_am = {0: pltpu.async_copy(am_hbm.at[wid], am2_v.at[0], am_s0)}
    pend_vals = pltpu.async_copy(vals_hbm.at[wid], vals_v, vals_s)
    zero_range(0, P // _L)

    for j in range(nplanes):
      buf = j % 2
      plane = j * NW + wid
      pend_am[buf].wait()
      if j + 1 < nplanes:
        nbuf = 1 - buf
        pend_am[nbuf] = pltpu.async_copy(
            am_hbm.at[(j + 1) * NW + wid], am2_v.at[nbuf], am_sems[nbuf])
      pend_vals.wait()

      @plsc.parallel_loop(0, HW, _L, unroll=8)
      def scat(i, _buf=buf):
        am = am2_v[_buf, pl.ds(i, _L)]
        v = vals_v[pl.ds(i, _L)]
        q = (jnp.right_shift(am, 5).astype(jnp.float32) * third).astype(
            jnp.int32)
        p = jnp.bitwise_and(q, P - 1)
        plsc.addupdate_scatter(acc_v, [p], v)

      if j + 1 < nplanes:
        pend_vals = pltpu.async_copy(
            vals_hbm.at[(j + 1) * NW + wid], vals_v, vals_s)

      out_d = [
          pltpu.async_copy(acc_v.at[pl.ds(k * CH, CH)],
                           out_hbm.at[plane, pl.ds(k * CH, CH)], out_sems[k])
          for k in range(2)
      ]
      for k in range(_OUT_CHUNKS):
        if k + 2 < _OUT_CHUNKS:
          out_d.append(
              pltpu.async_copy(acc_v.at[pl.ds((k + 2) * CH, CH)],
                               out_hbm.at[plane, pl.ds((k + 2) * CH, CH)],
                               out_sems[k + 2]))
        out_d[k].wait()
        if j + 1 < nplanes:
          zero_range(k * CH, CH // _L)

  return scatter_planes


_BK = 2048
_PK = 2048


def _in_body(u_ref, a_ref, vt_ref, at_ref):
  vt_ref[...] = u_ref[0].T
  at_ref[...] = a_ref[0].T


def _make_in_t(B, HW, C, b):
  return pl.pallas_call(
      _in_body,
      grid=(HW // _BK,),
      in_specs=[
          pl.BlockSpec((1, _BK, C), lambda g, _b=b: (_b, g, 0)),
          pl.BlockSpec((1, _BK, C), lambda g, _b=b: (_b, g, 0)),
      ],
      out_specs=[
          pl.BlockSpec((C, _BK), lambda g: (0, g)),
          pl.BlockSpec((C, _BK), lambda g: (0, g)),
      ],
      out_shape=[
          jax.ShapeDtypeStruct((C, HW), jnp.float32),
          jax.ShapeDtypeStruct((C, HW), jnp.int32),
      ],
  )


def _out_body_first(s_ref, o_ref):
  o_ref[...] = s_ref[...].T[None]


def _out_body(big_ref, s_ref, o_ref):
  del big_ref
  o_ref[...] = s_ref[...].T[None]


def _make_out_t(B, P, C, b):
  out_spec = pl.BlockSpec((1, _PK, C), lambda g, _b=b: (_b, g, 0))
  src_spec = pl.BlockSpec((C, _PK), lambda g: (0, g))
  out_shape = jax.ShapeDtypeStruct((B, P, C), jnp.float32)
  if b == 0:
    return pl.pallas_call(
        _out_body_first,
        grid=(P // _PK,),
        in_specs=[src_spec],
        out_specs=out_spec,
        out_shape=out_shape,
    )
  return pl.pallas_call(
      _out_body,
      grid=(P // _PK,),
      in_specs=[pl.BlockSpec(memory_space=pl.ANY), src_spec],
      out_specs=out_spec,
      out_shape=out_shape,
      input_output_aliases={0: 0},
  )


def kernel(updates, argmax):
  B, H, W, C = updates.shape
  OH, OW = 2 * H, 2 * W
  HW = H * W
  P = OH * OW

  info = plsc.get_sparse_core_info()
  NW = info.num_cores * info.num_subcores

  vals_t = jnp.transpose(updates.reshape(B, HW, C), (0, 2, 1)).reshape(B * C, HW)
  am_t = jnp.transpose(argmax.reshape(B, HW, C), (0, 2, 1)).reshape(B * C, HW)

  out_t = _make_scatter(B * C, HW, P, NW)(vals_t, am_t)

  return jnp.transpose(out_t.reshape(B, C, P), (0, 2, 1)).reshape(B, OH, OW, C)


# speedup vs baseline: 2.2827x; 1.0005x over previous
import functools

import jax
import jax.numpy as jnp
import numpy as np
from jax import lax
from jax.experimental import pallas as pl
from jax.experimental.pallas import tpu as pltpu
from jax.experimental.pallas import tpu_sc as plsc

_L = 16
_OUT_CHUNKS = 4


def _make_scatter(BC, HW, P, NW):
  nplanes = BC // NW
  mesh = plsc.VectorSubcoreMesh(core_axis_name="c", subcore_axis_name="s")
  NC = mesh.num_cores
  CH = P // _OUT_CHUNKS

  @functools.partial(
      pl.kernel,
      out_type=jax.ShapeDtypeStruct((BC, P), jnp.float32),
      mesh=mesh,
      compiler_params=pltpu.CompilerParams(needs_layout_passes=False),
      scratch_types=[
          pltpu.VMEM((P,), jnp.float32),
          pltpu.VMEM((2, HW), jnp.int32),
          pltpu.VMEM((HW,), jnp.float32),
          pltpu.SemaphoreType.DMA,
          pltpu.SemaphoreType.DMA,
          pltpu.SemaphoreType.DMA,
          pltpu.SemaphoreType.DMA,
          pltpu.SemaphoreType.DMA,
          pltpu.SemaphoreType.DMA,
          pltpu.SemaphoreType.DMA,
      ],
  )
  def scatter_planes(vals_hbm, am_hbm, out_hbm, acc_v, am2_v, vals_v,
                     am_s0, am_s1, vals_s, o_s0, o_s1, o_s2, o_s3):
    wid = lax.axis_index("s") * NC + lax.axis_index("c")
    am_sems = (am_s0, am_s1)
    out_sems = (o_s0, o_s1, o_s2, o_s3)

    third = jnp.float32(1.0 / 3.0)
    zeros = jnp.zeros((_L,), jnp.float32)

    def zero_range(base, nvec):
      def zb(i, c):
        acc_v[pl.ds(base + i * _L, _L)] = zeros
        return c

      lax.fori_loop(0, nvec, zb, 0, unroll=8)

    pend_am = {0: pltpu.async_copy(am_hbm.at[wid], am2_v.at[0], am_s0)}
    pend_vals = pltpu.async_copy(vals_hbm.at[wid], vals_v, vals_s)
    zero_range(0, P // _L)

    for j in range(nplanes):
      buf = j % 2
      plane = j * NW + wid
      pend_am[buf].wait()
      if j + 1 < nplanes:
        nbuf = 1 - buf
        pend_am[nbuf] = pltpu.async_copy(
            am_hbm.at[(j + 1) * NW + wid], am2_v.at[nbuf], am_sems[nbuf])
      pend_vals.wait()

      @plsc.parallel_loop(0, HW, _L, unroll=8)
      def scat(i, _buf=buf):
        am = am2_v[_buf, pl.ds(i, _L)]
        v = vals_v[pl.ds(i, _L)]
        q = (jnp.right_shift(am, 5).astype(jnp.float32) * third).astype(
            jnp.int32)
        p = jnp.bitwise_and(q, P - 1)
        plsc.addupdate_scatter(acc_v, [p], v)

      if j + 1 < nplanes:
        pend_vals = pltpu.async_copy(
            vals_hbm.at[(j + 1) * NW + wid], vals_v, vals_s)

      out_d = [
          pltpu.async_copy(acc_v.at[pl.ds(k * CH, CH)],
                           out_hbm.at[plane, pl.ds(k * CH, CH)], out_sems[k])
          for k in range(2)
      ]
      for k in range(_OUT_CHUNKS):
        if k + 2 < _OUT_CHUNKS:
          out_d.append(
              pltpu.async_copy(acc_v.at[pl.ds((k + 2) * CH, CH)],
                               out_hbm.at[plane, pl.ds((k + 2) * CH, CH)],
                               out_sems[k + 2]))
        out_d[k].wait()
        if j + 1 < nplanes:
          zero_range(k * CH, CH // _L)

  return scatter_planes


def kernel(updates, argmax):
  B, H, W, C = updates.shape
  OH, OW = 2 * H, 2 * W
  HW = H * W
  P = OH * OW

  info = plsc.get_sparse_core_info()
  NW = info.num_cores * info.num_subcores

  vals_t = jnp.transpose(updates.reshape(B, HW, C), (0, 2, 1)).reshape(B * C, HW)
  am_t = jnp.transpose(argmax.reshape(B, HW, C), (0, 2, 1)).reshape(B * C, HW)

  out_t = _make_scatter(B * C, HW, P, NW)(vals_t, am_t)

  return jnp.transpose(out_t.reshape(B, C, P), (0, 2, 1)).reshape(B, OH, OW, C)
